# baseline (speedup 1.0000x reference)
import jax
import jax.numpy as jnp
from jax import lax
from jax.experimental import pallas as pl


def _gcn_kernel(x_ref, wt_ref, b_ref, adj_ref, out_ref):
    a = adj_ref[...]
    g = jnp.dot(a, x_ref[...],
                preferred_element_type=jnp.float32,
                precision=lax.Precision.DEFAULT)
    r = jnp.sum(a, axis=1, keepdims=True)
    out = jnp.dot(g, wt_ref[...],
                  preferred_element_type=jnp.float32,
                  precision=lax.Precision.DEFAULT) + r * b_ref[...]
    out_ref[...] = jnp.maximum(out, 0.0)


def _pick_tile(m, candidates):
    for c in candidates:
        if m % c == 0:
            return c
    return m


def kernel(x, adj, W, b):
    n_nodes, d_in = x.shape
    d_out = W.shape[0]
    m_rows = adj.shape[0]

    wt = W.T
    b2 = b.reshape(1, d_out)

    tm = _pick_tile(m_rows, (400, 200, 8, 1))
    out = pl.pallas_call(
        _gcn_kernel,
        grid=(m_rows // tm,),
        in_specs=[
            pl.BlockSpec((n_nodes, d_in), lambda i: (0, 0)),
            pl.BlockSpec((d_in, d_out), lambda i: (0, 0)),
            pl.BlockSpec((1, d_out), lambda i: (0, 0)),
            pl.BlockSpec((tm, n_nodes), lambda i: (i, 0)),
        ],
        out_specs=pl.BlockSpec((tm, d_out), lambda i: (i, 0)),
        out_shape=jax.ShapeDtypeStruct((m_rows, d_out), jnp.float32),
    )(x, wt, b2, adj)
    return out
